# baseline (device time: 43205 ns/iter reference)
import jax
import jax.numpy as jnp
from jax import lax
from jax.experimental import pallas as pl
from jax.experimental.pallas import tpu as pltpu

N_DEV = 4
FIX_STEPS = 32

_OUTER = (((1,), (1,)), ((0,), (0,)))
_INNER = (((2,), (1,)), ((0,), (0,)))


def kernel(x, A, B, C):
    Bb, S, D = x.shape
    N = A.shape[-1]
    AT = A.T

    def body(x_ref, at_ref, b_ref, c_ref, out_ref,
             hend_ref, carry_ref, send_sem, recv_sem):
        my = lax.axis_index("i")
        left = (my - 1) % N_DEV
        right = (my + 1) % N_DEV

        barrier = pltpu.get_barrier_semaphore()
        for nbr in (left, right):
            pl.semaphore_signal(
                barrier, inc=1,
                device_id=(nbr,), device_id_type=pl.DeviceIdType.MESH,
            )
        pl.semaphore_wait(barrier, 2)

        dAT = jnp.exp(at_ref[:, :])[None].astype(jnp.bfloat16)

        h = jnp.zeros((Bb, N, D), jnp.bfloat16)
        for t in range(S):
            xt = x_ref[:, t:t + 1, :].astype(jnp.bfloat16)
            bt = b_ref[:, t:t + 1, :].astype(jnp.bfloat16)
            ct = c_ref[:, t:t + 1, :].astype(jnp.bfloat16)
            h = h * dAT + lax.dot_general(
                bt, xt, _OUTER, preferred_element_type=jnp.float32,
            ).astype(jnp.bfloat16)
            out_ref[:, t:t + 1, :] = lax.dot_general(
                ct, h, _INNER, preferred_element_type=jnp.float32)
        hend_ref[...] = h

        rdma = pltpu.make_async_remote_copy(
            src_ref=hend_ref,
            dst_ref=carry_ref,
            send_sem=send_sem,
            recv_sem=recv_sem,
            device_id=(right,),
            device_id_type=pl.DeviceIdType.MESH,
        )
        rdma.start()
        rdma.wait()

        @pl.when(my != 0)
        def _fixup():
            g = carry_ref[...]
            for t in range(FIX_STEPS):
                g = g * dAT
                ct = c_ref[:, t:t + 1, :].astype(jnp.bfloat16)
                dy = lax.dot_general(
                    ct, g, _INNER, preferred_element_type=jnp.float32)
                out_ref[:, t:t + 1, :] = out_ref[:, t:t + 1, :] + dy

    return pl.pallas_call(
        body,
        out_shape=jax.ShapeDtypeStruct((Bb, S, D), jnp.float32),
        in_specs=[pl.BlockSpec(memory_space=pltpu.VMEM)] * 4,
        out_specs=pl.BlockSpec(memory_space=pltpu.VMEM),
        scratch_shapes=[
            pltpu.VMEM((Bb, N, D), jnp.bfloat16),
            pltpu.VMEM((Bb, N, D), jnp.bfloat16),
            pltpu.SemaphoreType.DMA,
            pltpu.SemaphoreType.DMA,
        ],
        compiler_params=pltpu.CompilerParams(collective_id=0),
    )(x, AT, B, C)


# device time: 20826 ns/iter; 2.0746x vs baseline; 2.0746x over previous
import jax
import jax.numpy as jnp
from jax import lax
from jax.experimental import pallas as pl
from jax.experimental.pallas import tpu as pltpu

N_DEV = 4
FIX_STEPS = 32
T = 8

_BATCH_DOT = (((2,), (1,)), ((0,), (0,)))


def kernel(x, A, B, C):
    Bb, S, D = x.shape
    N = A.shape[-1]
    AT = A.T
    nblk = S // T
    eye = jnp.eye(T, dtype=B.dtype)

    Br = B.reshape(Bb, nblk, T, N)
    B_bd = (Br[:, :, :, :, None] * eye[None, None, :, None, :]).reshape(
        Bb, nblk * T * N, T)
    Cr = C.reshape(Bb, nblk, T, N)
    C_bd = (Cr[:, :, :, None, :] * eye[None, None, :, :, None]).reshape(
        Bb, S, T * N)

    def body(x_ref, at_ref, bbd_ref, cbd_ref, out_ref,
             hend_ref, carry_ref, send_sem, recv_sem):
        my = lax.axis_index("i")
        left = (my - 1) % N_DEV
        right = (my + 1) % N_DEV

        barrier = pltpu.get_barrier_semaphore()
        for nbr in (left, right):
            pl.semaphore_signal(
                barrier, inc=1,
                device_id=(nbr,), device_id_type=pl.DeviceIdType.MESH,
            )
        pl.semaphore_wait(barrier, 2)

        dAT = jnp.exp(at_ref[:, :])[None]

        h = jnp.zeros((Bb, N, D), jnp.float32)
        for k in range(nblk):
            xblk = x_ref[:, k * T:(k + 1) * T, :]
            bbd = bbd_ref[:, k * T * N:(k + 1) * T * N, :]
            cbd = cbd_ref[:, k * T:(k + 1) * T, :]
            P = lax.dot_general(
                bbd, xblk, _BATCH_DOT, preferred_element_type=jnp.float32)
            hs = []
            for j in range(T):
                h = h * dAT + P[:, j * N:(j + 1) * N, :]
                hs.append(h)
            hstack = jnp.concatenate(hs, axis=1)
            out_ref[:, k * T:(k + 1) * T, :] = lax.dot_general(
                cbd, hstack, _BATCH_DOT, preferred_element_type=jnp.float32)
        hend_ref[...] = h

        rdma = pltpu.make_async_remote_copy(
            src_ref=hend_ref,
            dst_ref=carry_ref,
            send_sem=send_sem,
            recv_sem=recv_sem,
            device_id=(right,),
            device_id_type=pl.DeviceIdType.MESH,
        )
        rdma.start()
        rdma.wait()

        @pl.when(my != 0)
        def _fixup():
            g = carry_ref[...]
            for k in range(FIX_STEPS // T):
                cbd = cbd_ref[:, k * T:(k + 1) * T, :]
                gs = []
                for j in range(T):
                    g = g * dAT
                    gs.append(g)
                gstack = jnp.concatenate(gs, axis=1)
                dy = lax.dot_general(
                    cbd, gstack, _BATCH_DOT,
                    preferred_element_type=jnp.float32)
                out_ref[:, k * T:(k + 1) * T, :] = (
                    out_ref[:, k * T:(k + 1) * T, :] + dy)

    return pl.pallas_call(
        body,
        out_shape=jax.ShapeDtypeStruct((Bb, S, D), jnp.float32),
        in_specs=[pl.BlockSpec(memory_space=pltpu.VMEM)] * 4,
        out_specs=pl.BlockSpec(memory_space=pltpu.VMEM),
        scratch_shapes=[
            pltpu.VMEM((Bb, N, D), jnp.float32),
            pltpu.VMEM((Bb, N, D), jnp.float32),
            pltpu.SemaphoreType.DMA,
            pltpu.SemaphoreType.DMA,
        ],
        compiler_params=pltpu.CompilerParams(collective_id=0),
    )(x, AT, B_bd, C_bd)


# device time: 19763 ns/iter; 2.1862x vs baseline; 1.0538x over previous
import jax
import jax.numpy as jnp
from jax import lax
from jax.experimental import pallas as pl
from jax.experimental.pallas import tpu as pltpu

N_DEV = 4
FIX_STEPS = 32
T = 8

_BATCH_DOT = (((2,), (1,)), ((0,), (0,)))


def kernel(x, A, B, C):
    Bb, S, D = x.shape
    N = A.shape[-1]
    AT = A.T
    nblk = S // T
    eye = jnp.eye(T, dtype=B.dtype)

    Br = B.reshape(Bb, nblk, T, N)
    B_bd = (Br[:, :, :, :, None] * eye[None, None, :, None, :]).reshape(
        Bb, nblk * T * N, T).astype(jnp.bfloat16)
    Cr = C.reshape(Bb, nblk, T, N)
    C_bd = (Cr[:, :, :, None, :] * eye[None, None, :, :, None]).reshape(
        Bb, S, T * N).astype(jnp.bfloat16)

    def body(x_ref, at_ref, bbd_ref, cbd_ref, out_ref,
             hend_ref, carry_ref, send_sem, recv_sem):
        my = lax.axis_index("i")
        left = (my - 1) % N_DEV
        right = (my + 1) % N_DEV

        barrier = pltpu.get_barrier_semaphore()
        for nbr in (left, right):
            pl.semaphore_signal(
                barrier, inc=1,
                device_id=(nbr,), device_id_type=pl.DeviceIdType.MESH,
            )
        pl.semaphore_wait(barrier, 2)

        dAT = jnp.exp(at_ref[:, :])[None].astype(jnp.bfloat16)

        h = jnp.zeros((Bb, N, D), jnp.bfloat16)
        for k in range(nblk):
            xblk = x_ref[:, k * T:(k + 1) * T, :]
            bbd = bbd_ref[:, k * T * N:(k + 1) * T * N, :]
            cbd = cbd_ref[:, k * T:(k + 1) * T, :]
            P = lax.dot_general(
                bbd, xblk, _BATCH_DOT, preferred_element_type=jnp.float32,
            ).astype(jnp.bfloat16)
            hs = []
            for j in range(T):
                h = h * dAT + P[:, j * N:(j + 1) * N, :]
                hs.append(h)
            hstack = jnp.concatenate(hs, axis=1)
            out_ref[:, k * T:(k + 1) * T, :] = lax.dot_general(
                cbd, hstack, _BATCH_DOT, preferred_element_type=jnp.float32)
        hend_ref[...] = h

        rdma = pltpu.make_async_remote_copy(
            src_ref=hend_ref,
            dst_ref=carry_ref,
            send_sem=send_sem,
            recv_sem=recv_sem,
            device_id=(right,),
            device_id_type=pl.DeviceIdType.MESH,
        )
        rdma.start()
        rdma.wait()

        @pl.when(my != 0)
        def _fixup():
            g = carry_ref[...]
            for k in range(FIX_STEPS // T):
                cbd = cbd_ref[:, k * T:(k + 1) * T, :]
                gs = []
                for j in range(T):
                    g = g * dAT
                    gs.append(g)
                gstack = jnp.concatenate(gs, axis=1)
                dy = lax.dot_general(
                    cbd, gstack, _BATCH_DOT,
                    preferred_element_type=jnp.float32)
                out_ref[:, k * T:(k + 1) * T, :] = (
                    out_ref[:, k * T:(k + 1) * T, :] + dy)

    return pl.pallas_call(
        body,
        out_shape=jax.ShapeDtypeStruct((Bb, S, D), jnp.float32),
        in_specs=[pl.BlockSpec(memory_space=pltpu.VMEM)] * 4,
        out_specs=pl.BlockSpec(memory_space=pltpu.VMEM),
        scratch_shapes=[
            pltpu.VMEM((Bb, N, D), jnp.bfloat16),
            pltpu.VMEM((Bb, N, D), jnp.bfloat16),
            pltpu.SemaphoreType.DMA,
            pltpu.SemaphoreType.DMA,
        ],
        compiler_params=pltpu.CompilerParams(collective_id=0),
    )(x.astype(jnp.bfloat16), AT, B_bd, C_bd)


# device time: 19087 ns/iter; 2.2636x vs baseline; 1.0354x over previous
import jax
import jax.numpy as jnp
from jax import lax
from jax.experimental import pallas as pl
from jax.experimental.pallas import tpu as pltpu

N_DEV = 4
FIX_STEPS = 32
T = 8

_BATCH_DOT = (((2,), (1,)), ((0,), (0,)))


def kernel(x, A, B, C):
    Bb, S, D = x.shape
    N = A.shape[-1]
    AT = A.T
    nblk = S // T
    eye = jnp.eye(T, dtype=jnp.bfloat16)

    Br = B.astype(jnp.bfloat16).reshape(Bb, nblk, T, N)
    B_bd = (Br[:, :, :, :, None] * eye[None, None, :, None, :]).reshape(
        Bb, nblk * T * N, T)

    def body(x_ref, at_ref, bbd_ref, c_ref, out_ref,
             hend_ref, carry_ref, send_sem, recv_sem):
        my = lax.axis_index("i")
        left = (my - 1) % N_DEV
        right = (my + 1) % N_DEV

        barrier = pltpu.get_barrier_semaphore()
        for nbr in (left, right):
            pl.semaphore_signal(
                barrier, inc=1,
                device_id=(nbr,), device_id_type=pl.DeviceIdType.MESH,
            )
        pl.semaphore_wait(barrier, 2)

        dAT = jnp.exp(at_ref[:, :])[None].astype(jnp.bfloat16)

        c_all = c_ref[...].astype(jnp.bfloat16)
        ctile = jnp.concatenate([c_all] * T, axis=2)
        lane = lax.broadcasted_iota(jnp.int32, (Bb, S, T * N), 2)
        trow = lax.broadcasted_iota(jnp.int32, (Bb, S, T * N), 1)
        cbd_all = jnp.where((lane // N) == (trow % T), ctile,
                            jnp.bfloat16(0))

        h = jnp.zeros((Bb, N, D), jnp.bfloat16)
        for k in range(nblk):
            xblk = x_ref[:, k * T:(k + 1) * T, :].astype(jnp.bfloat16)
            bbd = bbd_ref[:, k * T * N:(k + 1) * T * N, :]
            cbd = cbd_all[:, k * T:(k + 1) * T, :]
            P = lax.dot_general(
                bbd, xblk, _BATCH_DOT, preferred_element_type=jnp.float32,
            ).astype(jnp.bfloat16)
            hs = []
            for j in range(T):
                h = h * dAT + P[:, j * N:(j + 1) * N, :]
                hs.append(h)
            hstack = jnp.concatenate(hs, axis=1)
            out_ref[:, k * T:(k + 1) * T, :] = lax.dot_general(
                cbd, hstack, _BATCH_DOT, preferred_element_type=jnp.float32)
        hend_ref[...] = h

        rdma = pltpu.make_async_remote_copy(
            src_ref=hend_ref,
            dst_ref=carry_ref,
            send_sem=send_sem,
            recv_sem=recv_sem,
            device_id=(right,),
            device_id_type=pl.DeviceIdType.MESH,
        )
        rdma.start()
        rdma.wait()

        @pl.when(my != 0)
        def _fixup():
            g = carry_ref[...]
            for k in range(FIX_STEPS // T):
                cbd = cbd_all[:, k * T:(k + 1) * T, :]
                gs = []
                for j in range(T):
                    g = g * dAT
                    gs.append(g)
                gstack = jnp.concatenate(gs, axis=1)
                dy = lax.dot_general(
                    cbd, gstack, _BATCH_DOT,
                    preferred_element_type=jnp.float32)
                out_ref[:, k * T:(k + 1) * T, :] = (
                    out_ref[:, k * T:(k + 1) * T, :] + dy)

    return pl.pallas_call(
        body,
        out_shape=jax.ShapeDtypeStruct((Bb, S, D), jnp.float32),
        in_specs=[pl.BlockSpec(memory_space=pltpu.VMEM)] * 4,
        out_specs=pl.BlockSpec(memory_space=pltpu.VMEM),
        scratch_shapes=[
            pltpu.VMEM((Bb, N, D), jnp.bfloat16),
            pltpu.VMEM((Bb, N, D), jnp.bfloat16),
            pltpu.SemaphoreType.DMA,
            pltpu.SemaphoreType.DMA,
        ],
        compiler_params=pltpu.CompilerParams(collective_id=0),
    )(x, AT, B_bd, C)


# device time: 14195 ns/iter; 3.0437x vs baseline; 1.3446x over previous
import jax
import jax.numpy as jnp
from jax import lax
from jax.experimental import pallas as pl
from jax.experimental.pallas import tpu as pltpu

N_DEV = 4
FIX_STEPS = 32
T = 8

_BATCH_DOT = (((2,), (1,)), ((0,), (0,)))


def kernel(x, A, B, C):
    Bb, S, D = x.shape
    N = A.shape[-1]
    AT = A.T
    nblk = S // T

    def body(x_ref, at_ref, b_ref, c_ref, out_ref,
             hend_ref, carry_ref, send_sem, recv_sem):
        my = lax.axis_index("i")
        left = (my - 1) % N_DEV
        right = (my + 1) % N_DEV

        barrier = pltpu.get_barrier_semaphore()
        for nbr in (left, right):
            pl.semaphore_signal(
                barrier, inc=1,
                device_id=(nbr,), device_id_type=pl.DeviceIdType.MESH,
            )
        pl.semaphore_wait(barrier, 2)

        dAT = jnp.exp(at_ref[:, :])[None].astype(jnp.bfloat16)

        c_all = c_ref[...].astype(jnp.bfloat16)
        ctile = jnp.concatenate([c_all] * T, axis=2)
        lane = lax.broadcasted_iota(jnp.int32, (Bb, S, T * N), 2)
        trow = lax.broadcasted_iota(jnp.int32, (Bb, S, T * N), 1)
        cbd_all = jnp.where((lane // N) == (trow % T), ctile,
                            jnp.bfloat16(0))

        rown = lax.broadcasted_iota(jnp.int32, (T * N, N), 0) % N
        coln = lax.broadcasted_iota(jnp.int32, (T * N, N), 1)
        bmask = (rown == coln)[None]
        ones_nd = jnp.ones((Bb, N, D), jnp.bfloat16)

        h = jnp.zeros((Bb, N, D), jnp.bfloat16)
        for k in range(nblk):
            xblk = x_ref[:, k * T:(k + 1) * T, :].astype(jnp.bfloat16)
            bblk = b_ref[:, k * T:(k + 1) * T, :].astype(jnp.bfloat16)
            cbd = cbd_all[:, k * T:(k + 1) * T, :]
            bup = jnp.broadcast_to(
                bblk[:, :, None, :], (Bb, T, N, N)).reshape(Bb, T * N, N)
            bsel = jnp.where(bmask, bup, jnp.bfloat16(0))
            bsplat = lax.dot_general(
                bsel, ones_nd, _BATCH_DOT, preferred_element_type=jnp.float32,
            ).astype(jnp.bfloat16)
            hs = []
            for j in range(T):
                h = (h * dAT
                     + bsplat[:, j * N:(j + 1) * N, :] * xblk[:, j:j + 1, :])
                hs.append(h)
            hstack = jnp.concatenate(hs, axis=1)
            out_ref[:, k * T:(k + 1) * T, :] = lax.dot_general(
                cbd, hstack, _BATCH_DOT, preferred_element_type=jnp.float32)
        hend_ref[...] = h

        rdma = pltpu.make_async_remote_copy(
            src_ref=hend_ref,
            dst_ref=carry_ref,
            send_sem=send_sem,
            recv_sem=recv_sem,
            device_id=(right,),
            device_id_type=pl.DeviceIdType.MESH,
        )
        rdma.start()
        rdma.wait()

        @pl.when(my != 0)
        def _fixup():
            g = carry_ref[...]
            for k in range(FIX_STEPS // T):
                cbd = cbd_all[:, k * T:(k + 1) * T, :]
                gs = []
                for j in range(T):
                    g = g * dAT
                    gs.append(g)
                gstack = jnp.concatenate(gs, axis=1)
                dy = lax.dot_general(
                    cbd, gstack, _BATCH_DOT,
                    preferred_element_type=jnp.float32)
                out_ref[:, k * T:(k + 1) * T, :] = (
                    out_ref[:, k * T:(k + 1) * T, :] + dy)

    return pl.pallas_call(
        body,
        out_shape=jax.ShapeDtypeStruct((Bb, S, D), jnp.float32),
        in_specs=[pl.BlockSpec(memory_space=pltpu.VMEM)] * 4,
        out_specs=pl.BlockSpec(memory_space=pltpu.VMEM),
        scratch_shapes=[
            pltpu.VMEM((Bb, N, D), jnp.bfloat16),
            pltpu.VMEM((Bb, N, D), jnp.bfloat16),
            pltpu.SemaphoreType.DMA,
            pltpu.SemaphoreType.DMA,
        ],
        compiler_params=pltpu.CompilerParams(collective_id=0),
    )(x, AT, B, C)


# device time: 13232 ns/iter; 3.2652x vs baseline; 1.0728x over previous
import jax
import jax.numpy as jnp
from jax import lax
from jax.experimental import pallas as pl
from jax.experimental.pallas import tpu as pltpu

N_DEV = 4
FIX_STEPS = 32
T = 16

_BATCH_DOT = (((2,), (1,)), ((0,), (0,)))


def kernel(x, A, B, C):
    Bb, S, D = x.shape
    N = A.shape[-1]
    AT = A.T
    nblk = S // T

    def body(x_ref, at_ref, b_ref, c_ref, out_ref,
             hend_ref, carry_ref, send_sem, recv_sem):
        my = lax.axis_index("i")
        left = (my - 1) % N_DEV
        right = (my + 1) % N_DEV

        barrier = pltpu.get_barrier_semaphore()
        for nbr in (left, right):
            pl.semaphore_signal(
                barrier, inc=1,
                device_id=(nbr,), device_id_type=pl.DeviceIdType.MESH,
            )
        pl.semaphore_wait(barrier, 2)

        dAT = jnp.exp(at_ref[:, :])[None].astype(jnp.bfloat16)

        c_all = c_ref[...].astype(jnp.bfloat16)
        ctile = jnp.concatenate([c_all] * T, axis=2)
        lane = lax.broadcasted_iota(jnp.int32, (Bb, S, T * N), 2)
        trow = lax.broadcasted_iota(jnp.int32, (Bb, S, T * N), 1)
        cbd_all = jnp.where((lane // N) == (trow % T), ctile,
                            jnp.bfloat16(0))

        rown = lax.broadcasted_iota(jnp.int32, (T * N, N), 0) % N
        coln = lax.broadcasted_iota(jnp.int32, (T * N, N), 1)
        bmask = (rown == coln)[None]
        ones_nd = jnp.ones((Bb, N, D), jnp.bfloat16)

        h = jnp.zeros((Bb, N, D), jnp.bfloat16)
        for k in range(nblk):
            xblk = x_ref[:, k * T:(k + 1) * T, :].astype(jnp.bfloat16)
            bblk = b_ref[:, k * T:(k + 1) * T, :].astype(jnp.bfloat16)
            cbd = cbd_all[:, k * T:(k + 1) * T, :]
            bup = jnp.broadcast_to(
                bblk[:, :, None, :], (Bb, T, N, N)).reshape(Bb, T * N, N)
            bsel = jnp.where(bmask, bup, jnp.bfloat16(0))
            bsplat = lax.dot_general(
                bsel, ones_nd, _BATCH_DOT, preferred_element_type=jnp.float32,
            ).astype(jnp.bfloat16)
            hs = []
            for j in range(T):
                h = (h * dAT
                     + bsplat[:, j * N:(j + 1) * N, :] * xblk[:, j:j + 1, :])
                hs.append(h)
            hstack = jnp.concatenate(hs, axis=1)
            out_ref[:, k * T:(k + 1) * T, :] = lax.dot_general(
                cbd, hstack, _BATCH_DOT, preferred_element_type=jnp.float32)
        hend_ref[...] = h

        rdma = pltpu.make_async_remote_copy(
            src_ref=hend_ref,
            dst_ref=carry_ref,
            send_sem=send_sem,
            recv_sem=recv_sem,
            device_id=(right,),
            device_id_type=pl.DeviceIdType.MESH,
        )
        rdma.start()
        rdma.wait()

        @pl.when(my != 0)
        def _fixup():
            g = carry_ref[...]
            for k in range(FIX_STEPS // T):
                cbd = cbd_all[:, k * T:(k + 1) * T, :]
                gs = []
                for j in range(T):
                    g = g * dAT
                    gs.append(g)
                gstack = jnp.concatenate(gs, axis=1)
                dy = lax.dot_general(
                    cbd, gstack, _BATCH_DOT,
                    preferred_element_type=jnp.float32)
                out_ref[:, k * T:(k + 1) * T, :] = (
                    out_ref[:, k * T:(k + 1) * T, :] + dy)

    return pl.pallas_call(
        body,
        out_shape=jax.ShapeDtypeStruct((Bb, S, D), jnp.float32),
        in_specs=[pl.BlockSpec(memory_space=pltpu.VMEM)] * 4,
        out_specs=pl.BlockSpec(memory_space=pltpu.VMEM),
        scratch_shapes=[
            pltpu.VMEM((Bb, N, D), jnp.bfloat16),
            pltpu.VMEM((Bb, N, D), jnp.bfloat16),
            pltpu.SemaphoreType.DMA,
            pltpu.SemaphoreType.DMA,
        ],
        compiler_params=pltpu.CompilerParams(collective_id=0),
    )(x, AT, B, C)


# device time: 11324 ns/iter; 3.8153x vs baseline; 1.1685x over previous
import jax
import jax.numpy as jnp
from jax import lax
from jax.experimental import pallas as pl
from jax.experimental.pallas import tpu as pltpu

N_DEV = 4
FIX_STEPS = 32
T = 32

_BATCH_DOT = (((2,), (1,)), ((0,), (0,)))


def kernel(x, A, B, C):
    Bb, S, D = x.shape
    N = A.shape[-1]
    AT = A.T
    nblk = S // T

    def body(x_ref, at_ref, b_ref, c_ref, out_ref,
             hend_ref, carry_ref, send_sem, recv_sem):
        my = lax.axis_index("i")
        left = (my - 1) % N_DEV
        right = (my + 1) % N_DEV

        barrier = pltpu.get_barrier_semaphore()
        for nbr in (left, right):
            pl.semaphore_signal(
                barrier, inc=1,
                device_id=(nbr,), device_id_type=pl.DeviceIdType.MESH,
            )
        pl.semaphore_wait(barrier, 2)

        dAT = jnp.exp(at_ref[:, :])[None].astype(jnp.bfloat16)

        c_all = c_ref[...].astype(jnp.bfloat16)
        ctile = jnp.concatenate([c_all] * T, axis=2)
        lane = lax.broadcasted_iota(jnp.int32, (Bb, S, T * N), 2)
        trow = lax.broadcasted_iota(jnp.int32, (Bb, S, T * N), 1)
        cbd_all = jnp.where((lane // N) == (trow % T), ctile,
                            jnp.bfloat16(0))

        rown = lax.broadcasted_iota(jnp.int32, (T * N, N), 0) % N
        coln = lax.broadcasted_iota(jnp.int32, (T * N, N), 1)
        bmask = (rown == coln)[None]
        ones_nd = jnp.ones((Bb, N, D), jnp.bfloat16)

        h = jnp.zeros((Bb, N, D), jnp.bfloat16)
        for k in range(nblk):
            xblk = x_ref[:, k * T:(k + 1) * T, :].astype(jnp.bfloat16)
            bblk = b_ref[:, k * T:(k + 1) * T, :].astype(jnp.bfloat16)
            cbd = cbd_all[:, k * T:(k + 1) * T, :]
            bup = jnp.broadcast_to(
                bblk[:, :, None, :], (Bb, T, N, N)).reshape(Bb, T * N, N)
            bsel = jnp.where(bmask, bup, jnp.bfloat16(0))
            bsplat = lax.dot_general(
                bsel, ones_nd, _BATCH_DOT, preferred_element_type=jnp.float32,
            ).astype(jnp.bfloat16)
            hs = []
            for j in range(T):
                h = (h * dAT
                     + bsplat[:, j * N:(j + 1) * N, :] * xblk[:, j:j + 1, :])
                hs.append(h)
            hstack = jnp.concatenate(hs, axis=1)
            out_ref[:, k * T:(k + 1) * T, :] = lax.dot_general(
                cbd, hstack, _BATCH_DOT, preferred_element_type=jnp.float32)
        hend_ref[...] = h

        rdma = pltpu.make_async_remote_copy(
            src_ref=hend_ref,
            dst_ref=carry_ref,
            send_sem=send_sem,
            recv_sem=recv_sem,
            device_id=(right,),
            device_id_type=pl.DeviceIdType.MESH,
        )
        rdma.start()
        rdma.wait()

        @pl.when(my != 0)
        def _fixup():
            g = carry_ref[...]
            for k in range(FIX_STEPS // T):
                cbd = cbd_all[:, k * T:(k + 1) * T, :]
                gs = []
                for j in range(T):
                    g = g * dAT
                    gs.append(g)
                gstack = jnp.concatenate(gs, axis=1)
                dy = lax.dot_general(
                    cbd, gstack, _BATCH_DOT,
                    preferred_element_type=jnp.float32)
                out_ref[:, k * T:(k + 1) * T, :] = (
                    out_ref[:, k * T:(k + 1) * T, :] + dy)

    return pl.pallas_call(
        body,
        out_shape=jax.ShapeDtypeStruct((Bb, S, D), jnp.float32),
        in_specs=[pl.BlockSpec(memory_space=pltpu.VMEM)] * 4,
        out_specs=pl.BlockSpec(memory_space=pltpu.VMEM),
        scratch_shapes=[
            pltpu.VMEM((Bb, N, D), jnp.bfloat16),
            pltpu.VMEM((Bb, N, D), jnp.bfloat16),
            pltpu.SemaphoreType.DMA,
            pltpu.SemaphoreType.DMA,
        ],
        compiler_params=pltpu.CompilerParams(collective_id=0),
    )(x, AT, B, C)
